# Initial kernel scaffold; baseline (speedup 1.0000x reference)
import jax
import jax.numpy as jnp
from jax.experimental import pallas as pl


def kernel(theta, CL_idxs, n_vars, n_clauses):
    raise NotImplementedError("write your pallas kernel here")


# speedup vs baseline: 1.0510x; 1.0510x over previous
import functools
import jax
import jax.numpy as jnp
from jax import lax
from jax.experimental import pallas as pl
from jax.experimental.pallas import tpu as pltpu
from jax.experimental.pallas import tpu_sc as plsc

D = 256
NV = 10000
NLITS = 2 * NV
NCL = 20000
E = 160000
EPS = 1e-6

SEG = 320
NSEG = 64
OUT_PAD = NSEG * SEG
K = 64
PAD_E = E + NSEG * K
DUMMY_ROW = SEG
ACC_ROWS = SEG + 8
SOFF_LEN = 528


@functools.lru_cache(maxsize=1)
def _make_sc_spmm():
    mesh = plsc.VectorSubcoreMesh(core_axis_name="c", subcore_axis_name="s")
    return pl.kernel(
        _sc_spmm_body,
        mesh=mesh,
        out_type=jax.ShapeDtypeStruct((OUT_PAD * D,), jnp.float32),
        scratch_types=[
            pltpu.VMEM((K,), jnp.int32),
            pltpu.VMEM((K,), jnp.int32),
            pltpu.VMEM((K, D), jnp.float32),
            pltpu.VMEM((ACC_ROWS * D,), jnp.float32),
            pltpu.VMEM((16,), jnp.int32),
            pltpu.SemaphoreType.DMA,
        ],
    )


def _sc_spmm(*args):
    return _make_sc_spmm()(*args)


def _sc_spmm_body(src_hbm, psrc_hbm, pdloc_hbm, soff_hbm, zeros_hbm, out_hbm,
                  idx_v, dloc_v, rows_v, acc, m16_v, sem):
    c = lax.axis_index("c")
    s = lax.axis_index("s")
    w = s * 2 + c
    for k in range(2):
        j = 2 * w + k
        pltpu.sync_copy(soff_hbm.at[pl.ds(pl.multiple_of(8 * j, 8), 16)],
                        m16_v)
        wv = m16_v[...]
        g0 = wv[0]
        g1 = wv[8]
        m = (g1 - g0) // K
        pltpu.sync_copy(zeros_hbm, acc)

        def chunk_body(i, carry):
            e0 = pl.multiple_of(g0 + i * K, K)
            pltpu.sync_copy(psrc_hbm.at[pl.ds(e0, K)], idx_v)
            pltpu.async_copy(src_hbm.at[idx_v], rows_v, sem).wait()
            pltpu.sync_copy(pdloc_hbm.at[pl.ds(e0, K)], dloc_v)
            for gg in range(K // 16):
                dvec = dloc_v[pl.ds(gg * 16, 16)]
                for i16 in range(16):
                    d = dvec[i16]
                    base = d * D
                    row = gg * 16 + i16
                    for jj in range(D // 16):
                        v = rows_v[row, pl.ds(jj * 16, 16)]
                        plsc.addupdate(
                            acc.at[pl.ds(pl.multiple_of(base + jj * 16, 16),
                                         16)], v)
            return carry

        lax.fori_loop(0, m, chunk_body, 0)
        pltpu.sync_copy(acc.at[pl.ds(0, SEG * D)],
                        out_hbm.at[pl.ds(j * (SEG * D), SEG * D)])


def _build_edge_layout(dst, src):
    order = jnp.argsort(dst)
    dsts = dst[order]
    srcs = src[order]
    seg = dsts // SEG
    cnt = jnp.bincount(seg, length=NSEG)
    goff = jnp.concatenate([jnp.zeros((1,), jnp.int32),
                            jnp.cumsum(cnt).astype(jnp.int32)])
    cnt_p = ((cnt + K - 1) // K) * K
    goff_p = jnp.concatenate([jnp.zeros((1,), jnp.int32),
                              jnp.cumsum(cnt_p).astype(jnp.int32)])
    pos = goff_p[seg] + (jnp.arange(E, dtype=jnp.int32) - goff[seg])
    psrc = jnp.zeros((PAD_E,), jnp.int32).at[pos].set(srcs)
    pdloc = jnp.full((PAD_E,), DUMMY_ROW, jnp.int32).at[pos].set(
        (dsts - seg * SEG).astype(jnp.int32))
    soff = jnp.zeros((SOFF_LEN,), jnp.int32).at[
        8 * jnp.arange(NSEG + 1)].set(goff_p)
    return psrc, pdloc, soff


BR = 1000
_NBC = NCL // BR
_NBL = NLITS // BR
_NBV = NV // BR


def _norm_rows(x):
    m = jnp.mean(x, axis=1, keepdims=True)
    xc = x - m
    var = jnp.sum(xc * xc, axis=1, keepdims=True) / (D - 1)
    return xc / (jnp.sqrt(var) + EPS)


def _cu_body(old_ref, msg_ref, w1a_ref, w1b_ref, b1_ref,
             w2_ref, b2_ref, w3_ref, b3_ref, out_ref):
    old = old_ref[...]
    x = (jnp.dot(old, w1a_ref[...], preferred_element_type=jnp.float32)
         + jnp.dot(msg_ref[...], w1b_ref[...], preferred_element_type=jnp.float32)
         + b1_ref[...])
    x = jnp.maximum(x, 0.0)
    x = jnp.maximum(jnp.dot(x, w2_ref[...], preferred_element_type=jnp.float32)
                    + b2_ref[...], 0.0)
    x = jnp.dot(x, w3_ref[...], preferred_element_type=jnp.float32) + b3_ref[...]
    out_ref[...] = _norm_rows(x) + old


def _lu_body(old_ref, msg_ref, flip_ref, w1a_ref, w1b_ref, w1c_ref, b1_ref,
             w2_ref, b2_ref, w3_ref, b3_ref, out_ref):
    old = old_ref[...]
    x = (jnp.dot(old, w1a_ref[...], preferred_element_type=jnp.float32)
         + jnp.dot(msg_ref[...], w1b_ref[...], preferred_element_type=jnp.float32)
         + jnp.dot(flip_ref[...], w1c_ref[...], preferred_element_type=jnp.float32)
         + b1_ref[...])
    x = jnp.maximum(x, 0.0)
    x = jnp.maximum(jnp.dot(x, w2_ref[...], preferred_element_type=jnp.float32)
                    + b2_ref[...], 0.0)
    x = jnp.dot(x, w3_ref[...], preferred_element_type=jnp.float32) + b3_ref[...]
    out_ref[...] = _norm_rows(x) + old


def _vs_body(la_ref, lb_ref, w1a_ref, w1b_ref, b1_ref,
             w2_ref, b2_ref, w3_ref, b3_ref, out_ref):
    x = (jnp.dot(la_ref[...], w1a_ref[...], preferred_element_type=jnp.float32)
         + jnp.dot(lb_ref[...], w1b_ref[...], preferred_element_type=jnp.float32)
         + b1_ref[...])
    x = jnp.maximum(x, 0.0)
    x = jnp.maximum(jnp.dot(x, w2_ref[...], preferred_element_type=jnp.float32)
                    + b2_ref[...], 0.0)
    out_ref[...] = (jnp.dot(x, w3_ref[...], preferred_element_type=jnp.float32)
                    + b3_ref[...])


def _w_spec():
    return pl.BlockSpec((D, D), lambda i: (0, 0))


def _b_spec():
    return pl.BlockSpec((1, D), lambda i: (0, 0))


def _row_spec(idx_fn=None):
    return pl.BlockSpec((BR, D), idx_fn or (lambda i: (i, 0)))


def _call_cu(C, msgs, w1a, w1b, b1, w2, b2, w3, b3):
    return pl.pallas_call(
        _cu_body,
        grid=(_NBC,),
        in_specs=[_row_spec(), _row_spec(), _w_spec(), _w_spec(), _b_spec(),
                  _w_spec(), _b_spec(), _w_spec(), _b_spec()],
        out_specs=_row_spec(),
        out_shape=jax.ShapeDtypeStruct((NCL, D), jnp.float32),
    )(C, msgs, w1a, w1b, b1, w2, b2, w3, b3)


def _call_lu(L, msgs, w1a, w1b, w1c, b1, w2, b2, w3, b3):
    flip_spec = pl.BlockSpec((BR, D), lambda i: ((i + _NBL // 2) % _NBL, 0))
    return pl.pallas_call(
        _lu_body,
        grid=(_NBL,),
        in_specs=[_row_spec(), _row_spec(), flip_spec,
                  _w_spec(), _w_spec(), _w_spec(), _b_spec(),
                  _w_spec(), _b_spec(), _w_spec(), _b_spec()],
        out_specs=_row_spec(),
        out_shape=jax.ShapeDtypeStruct((NLITS, D), jnp.float32),
    )(L, msgs, L, w1a, w1b, w1c, b1, w2, b2, w3, b3)


def _call_vs(L, w1a, w1b, b1, w2, b2, w3p, b3p):
    lb_spec = pl.BlockSpec((BR, D), lambda i: (i + _NBV, 0))
    return pl.pallas_call(
        _vs_body,
        grid=(_NBV,),
        in_specs=[_row_spec(), lb_spec, _w_spec(), _w_spec(), _b_spec(),
                  _w_spec(), _b_spec(),
                  pl.BlockSpec((D, 128), lambda i: (0, 0)),
                  pl.BlockSpec((1, 128), lambda i: (0, 0))],
        out_specs=pl.BlockSpec((BR, 128), lambda i: (i, 0)),
        out_shape=jax.ShapeDtypeStruct((NV, 128), jnp.float32),
    )(L, L, w1a, w1b, b1, w2, b2, w3p, b3p)


def kernel(theta, CL_idxs, n_vars, n_clauses):
    zero = (jnp.asarray(n_vars) - NV
            + jnp.asarray(n_clauses) - NCL).astype(jnp.float32)
    c_idx = CL_idxs[:, 0].astype(jnp.int32)
    l_idx = CL_idxs[:, 1].astype(jnp.int32)

    lc_layout = _build_edge_layout(c_idx, l_idx)
    cl_layout = _build_edge_layout(l_idx, c_idx)
    zeros_acc = jnp.zeros((ACC_ROWS * D,), jnp.float32)

    L = jnp.full((NLITS, D), 1.0, jnp.float32) * theta['L_init'] + zero
    C = jnp.full((NCL, D), 1.0, jnp.float32) * theta['C_init'] + zero

    lc_s = theta['LC']
    cl_s = theta['CL']

    for t in range(8):
        (w1, b1), (w2, b2), (w3, b3) = theta['Cu'][t]
        msgs = _sc_spmm(L, *lc_layout, zeros_acc).reshape(OUT_PAD, D)
        C = _call_cu(C, msgs, w1[:D], w1[D:] * lc_s, b1.reshape(1, D),
                     w2, b2.reshape(1, D), w3, b3.reshape(1, D))
        (w1, b1), (w2, b2), (w3, b3) = theta['Lu'][t]
        msgsl = _sc_spmm(C, *cl_layout, zeros_acc).reshape(OUT_PAD, D)
        L = _call_lu(L, msgsl, w1[:D], w1[D:2 * D] * cl_s, w1[2 * D:],
                     b1.reshape(1, D), w2, b2.reshape(1, D),
                     w3, b3.reshape(1, D))

    (w1, b1), (w2, b2), (w3, b3) = theta['Vs']
    w3p = jnp.zeros((D, 128), jnp.float32).at[:, :1].set(w3)
    b3p = jnp.zeros((1, 128), jnp.float32).at[0, :1].set(b3)
    out = _call_vs(L, w1[:D], w1[D:], b1.reshape(1, D),
                   w2, b2.reshape(1, D), w3p, b3p)
    return out[:, 0]


# speedup vs baseline: 1.2495x; 1.1889x over previous
import functools
import jax
import jax.numpy as jnp
from jax import lax
from jax.experimental import pallas as pl
from jax.experimental.pallas import tpu as pltpu
from jax.experimental.pallas import tpu_sc as plsc

D = 256
NV = 10000
NLITS = 2 * NV
NCL = 20000
E = 160000
EPS = 1e-6

SEG = 320
NSEG = 64
OUT_PAD = NSEG * SEG
K = 64
PAD_E = E + NSEG * K
DUMMY_ROW = SEG
ACC_ROWS = SEG + 8
SOFF_LEN = 528


@functools.lru_cache(maxsize=1)
def _make_sc_spmm():
    mesh = plsc.VectorSubcoreMesh(core_axis_name="c", subcore_axis_name="s")
    return pl.kernel(
        _sc_spmm_body,
        mesh=mesh,
        out_type=jax.ShapeDtypeStruct((OUT_PAD * D,), jnp.float32),
        scratch_types=[
            pltpu.VMEM((2 * K,), jnp.int32),
            pltpu.VMEM((2 * K,), jnp.int32),
            pltpu.VMEM((K, D), jnp.float32),
            pltpu.VMEM((K, D), jnp.float32),
            pltpu.VMEM((ACC_ROWS * D,), jnp.float32),
            pltpu.VMEM((16,), jnp.int32),
            pltpu.SemaphoreType.DMA,
            pltpu.SemaphoreType.DMA,
        ],
    )


def _sc_spmm(*args):
    return _make_sc_spmm()(*args)


def _sc_spmm_body(src_hbm, pmix_hbm, soff_hbm, zeros_hbm, out_hbm,
                  mix0, mix1, rows0, rows1, acc, m16_v, sem0, sem1):
    c = lax.axis_index("c")
    s = lax.axis_index("s")
    w = s * 2 + c

    def load_mix(ch, mix_ref):
        off = pl.multiple_of(2 * K * ch, 2 * K)
        pltpu.sync_copy(pmix_hbm.at[pl.ds(off, 2 * K)], mix_ref)

    def start_gather(mix_ref, rows_ref, sem):
        pltpu.async_copy(src_hbm.at[mix_ref.at[pl.ds(0, K)]], rows_ref, sem)

    def wait_gather(mix_ref, rows_ref, sem):
        pltpu.make_async_copy(src_hbm.at[mix_ref.at[pl.ds(0, K)]],
                              rows_ref, sem).wait()

    def process(mix_ref, rows_ref):
        def grp_body(gg, carry):
            dvec = mix_ref[pl.ds(pl.multiple_of(K + gg * 16, 16), 16)]
            for i16 in range(16):
                d = dvec[i16]
                base = d * D
                row = gg * 16 + i16
                for jj in range(D // 16):
                    v = rows_ref[row, pl.ds(jj * 16, 16)]
                    plsc.addupdate(
                        acc.at[pl.ds(pl.multiple_of(base + jj * 16, 16),
                                     16)], v)
            return carry

        lax.fori_loop(0, K // 16, grp_body, 0)

    for k in range(2):
        j = 2 * w + k
        pltpu.sync_copy(soff_hbm.at[pl.ds(pl.multiple_of(8 * j, 8), 16)],
                        m16_v)
        wv = m16_v[...]
        ch0 = wv[0] // K
        m = (wv[8] - wv[0]) // K
        pltpu.sync_copy(zeros_hbm, acc)

        @pl.when(m > 0)
        def _():
            load_mix(ch0, mix0)
            start_gather(mix0, rows0, sem0)

        def pair_body(ii, carry):
            a = 2 * ii

            @pl.when(a + 1 < m)
            def _():
                load_mix(ch0 + a + 1, mix1)
                start_gather(mix1, rows1, sem1)

            wait_gather(mix0, rows0, sem0)
            process(mix0, rows0)

            @pl.when(a + 1 < m)
            def _():
                @pl.when(a + 2 < m)
                def _():
                    load_mix(ch0 + a + 2, mix0)
                    start_gather(mix0, rows0, sem0)

                wait_gather(mix1, rows1, sem1)
                process(mix1, rows1)

            return carry

        lax.fori_loop(0, (m + 1) // 2, pair_body, 0)
        pltpu.sync_copy(acc.at[pl.ds(0, SEG * D)],
                        out_hbm.at[pl.ds(j * (SEG * D), SEG * D)])


def _build_edge_layout(dst, src):
    order = jnp.argsort(dst)
    dsts = dst[order]
    srcs = src[order]
    seg = dsts // SEG
    cnt = jnp.bincount(seg, length=NSEG)
    goff = jnp.concatenate([jnp.zeros((1,), jnp.int32),
                            jnp.cumsum(cnt).astype(jnp.int32)])
    cnt_p = ((cnt + K - 1) // K) * K
    goff_p = jnp.concatenate([jnp.zeros((1,), jnp.int32),
                              jnp.cumsum(cnt_p).astype(jnp.int32)])
    pos = goff_p[seg] + (jnp.arange(E, dtype=jnp.int32) - goff[seg])
    psrc = jnp.zeros((PAD_E,), jnp.int32).at[pos].set(srcs)
    pdloc = jnp.full((PAD_E,), DUMMY_ROW, jnp.int32).at[pos].set(
        (dsts - seg * SEG).astype(jnp.int32))
    pmix = jnp.concatenate([psrc.reshape(-1, K), pdloc.reshape(-1, K)],
                           axis=1).reshape(-1)
    soff = jnp.zeros((SOFF_LEN,), jnp.int32).at[
        8 * jnp.arange(NSEG + 1)].set(goff_p)
    return pmix, soff


BR = 1000
_NBC = NCL // BR
_NBL = NLITS // BR
_NBV = NV // BR


def _norm_rows(x):
    m = jnp.mean(x, axis=1, keepdims=True)
    xc = x - m
    var = jnp.sum(xc * xc, axis=1, keepdims=True) / (D - 1)
    return xc / (jnp.sqrt(var) + EPS)


def _cu_body(old_ref, msg_ref, w1a_ref, w1b_ref, b1_ref,
             w2_ref, b2_ref, w3_ref, b3_ref, out_ref):
    old = old_ref[...]
    x = (jnp.dot(old, w1a_ref[...], preferred_element_type=jnp.float32)
         + jnp.dot(msg_ref[...], w1b_ref[...], preferred_element_type=jnp.float32)
         + b1_ref[...])
    x = jnp.maximum(x, 0.0)
    x = jnp.maximum(jnp.dot(x, w2_ref[...], preferred_element_type=jnp.float32)
                    + b2_ref[...], 0.0)
    x = jnp.dot(x, w3_ref[...], preferred_element_type=jnp.float32) + b3_ref[...]
    out_ref[...] = _norm_rows(x) + old


def _lu_body(old_ref, msg_ref, flip_ref, w1a_ref, w1b_ref, w1c_ref, b1_ref,
             w2_ref, b2_ref, w3_ref, b3_ref, out_ref):
    old = old_ref[...]
    x = (jnp.dot(old, w1a_ref[...], preferred_element_type=jnp.float32)
         + jnp.dot(msg_ref[...], w1b_ref[...], preferred_element_type=jnp.float32)
         + jnp.dot(flip_ref[...], w1c_ref[...], preferred_element_type=jnp.float32)
         + b1_ref[...])
    x = jnp.maximum(x, 0.0)
    x = jnp.maximum(jnp.dot(x, w2_ref[...], preferred_element_type=jnp.float32)
                    + b2_ref[...], 0.0)
    x = jnp.dot(x, w3_ref[...], preferred_element_type=jnp.float32) + b3_ref[...]
    out_ref[...] = _norm_rows(x) + old


def _vs_body(la_ref, lb_ref, w1a_ref, w1b_ref, b1_ref,
             w2_ref, b2_ref, w3_ref, b3_ref, out_ref):
    x = (jnp.dot(la_ref[...], w1a_ref[...], preferred_element_type=jnp.float32)
         + jnp.dot(lb_ref[...], w1b_ref[...], preferred_element_type=jnp.float32)
         + b1_ref[...])
    x = jnp.maximum(x, 0.0)
    x = jnp.maximum(jnp.dot(x, w2_ref[...], preferred_element_type=jnp.float32)
                    + b2_ref[...], 0.0)
    out_ref[...] = (jnp.dot(x, w3_ref[...], preferred_element_type=jnp.float32)
                    + b3_ref[...])


def _w_spec():
    return pl.BlockSpec((D, D), lambda i: (0, 0))


def _b_spec():
    return pl.BlockSpec((1, D), lambda i: (0, 0))


def _row_spec(idx_fn=None):
    return pl.BlockSpec((BR, D), idx_fn or (lambda i: (i, 0)))


def _call_cu(C, msgs, w1a, w1b, b1, w2, b2, w3, b3):
    return pl.pallas_call(
        _cu_body,
        grid=(_NBC,),
        in_specs=[_row_spec(), _row_spec(), _w_spec(), _w_spec(), _b_spec(),
                  _w_spec(), _b_spec(), _w_spec(), _b_spec()],
        out_specs=_row_spec(),
        out_shape=jax.ShapeDtypeStruct((NCL, D), jnp.float32),
    )(C, msgs, w1a, w1b, b1, w2, b2, w3, b3)


def _call_lu(L, msgs, w1a, w1b, w1c, b1, w2, b2, w3, b3):
    flip_spec = pl.BlockSpec((BR, D), lambda i: ((i + _NBL // 2) % _NBL, 0))
    return pl.pallas_call(
        _lu_body,
        grid=(_NBL,),
        in_specs=[_row_spec(), _row_spec(), flip_spec,
                  _w_spec(), _w_spec(), _w_spec(), _b_spec(),
                  _w_spec(), _b_spec(), _w_spec(), _b_spec()],
        out_specs=_row_spec(),
        out_shape=jax.ShapeDtypeStruct((NLITS, D), jnp.float32),
    )(L, msgs, L, w1a, w1b, w1c, b1, w2, b2, w3, b3)


def _call_vs(L, w1a, w1b, b1, w2, b2, w3p, b3p):
    lb_spec = pl.BlockSpec((BR, D), lambda i: (i + _NBV, 0))
    return pl.pallas_call(
        _vs_body,
        grid=(_NBV,),
        in_specs=[_row_spec(), lb_spec, _w_spec(), _w_spec(), _b_spec(),
                  _w_spec(), _b_spec(),
                  pl.BlockSpec((D, 128), lambda i: (0, 0)),
                  pl.BlockSpec((1, 128), lambda i: (0, 0))],
        out_specs=pl.BlockSpec((BR, 128), lambda i: (i, 0)),
        out_shape=jax.ShapeDtypeStruct((NV, 128), jnp.float32),
    )(L, L, w1a, w1b, b1, w2, b2, w3p, b3p)


def kernel(theta, CL_idxs, n_vars, n_clauses):
    zero = (jnp.asarray(n_vars) - NV
            + jnp.asarray(n_clauses) - NCL).astype(jnp.float32)
    c_idx = CL_idxs[:, 0].astype(jnp.int32)
    l_idx = CL_idxs[:, 1].astype(jnp.int32)

    lc_layout = _build_edge_layout(c_idx, l_idx)
    cl_layout = _build_edge_layout(l_idx, c_idx)
    zeros_acc = jnp.zeros((ACC_ROWS * D,), jnp.float32)

    L = jnp.full((NLITS, D), 1.0, jnp.float32) * theta['L_init'] + zero
    C = jnp.full((NCL, D), 1.0, jnp.float32) * theta['C_init'] + zero

    lc_s = theta['LC']
    cl_s = theta['CL']

    for t in range(8):
        (w1, b1), (w2, b2), (w3, b3) = theta['Cu'][t]
        msgs = _sc_spmm(L, *lc_layout, zeros_acc).reshape(OUT_PAD, D)
        C = _call_cu(C, msgs, w1[:D], w1[D:] * lc_s, b1.reshape(1, D),
                     w2, b2.reshape(1, D), w3, b3.reshape(1, D))
        (w1, b1), (w2, b2), (w3, b3) = theta['Lu'][t]
        msgsl = _sc_spmm(C, *cl_layout, zeros_acc).reshape(OUT_PAD, D)
        L = _call_lu(L, msgsl, w1[:D], w1[D:2 * D] * cl_s, w1[2 * D:],
                     b1.reshape(1, D), w2, b2.reshape(1, D),
                     w3, b3.reshape(1, D))

    (w1, b1), (w2, b2), (w3, b3) = theta['Vs']
    w3p = jnp.zeros((D, 128), jnp.float32).at[:, :1].set(w3)
    b3p = jnp.zeros((1, 128), jnp.float32).at[0, :1].set(b3)
    out = _call_vs(L, w1[:D], w1[D:], b1.reshape(1, D),
                   w2, b2.reshape(1, D), w3p, b3p)
    return out[:, 0]


# speedup vs baseline: 1.5119x; 1.2100x over previous
import functools
import jax
import jax.numpy as jnp
from jax import lax
from jax.experimental import pallas as pl
from jax.experimental.pallas import tpu as pltpu
from jax.experimental.pallas import tpu_sc as plsc

D = 256
NV = 10000
NLITS = 2 * NV
NCL = 20000
E = 160000
EPS = 1e-6

SEG = 320
NSEG = 64
OUT_PAD = NSEG * SEG
K = 64
PAD_E = E + NSEG * K
DUMMY_ROW = SEG
ACC_ROWS = SEG + 8
SOFF_LEN = 528


@functools.lru_cache(maxsize=1)
def _make_sc_spmm():
    mesh = plsc.VectorSubcoreMesh(core_axis_name="c", subcore_axis_name="s")
    return pl.kernel(
        _sc_spmm_body,
        mesh=mesh,
        out_type=jax.ShapeDtypeStruct((OUT_PAD * D,), jnp.float32),
        scratch_types=[
            pltpu.VMEM((2 * K,), jnp.int32),
            pltpu.VMEM((2 * K,), jnp.int32),
            pltpu.VMEM((K, D), jnp.float32),
            pltpu.VMEM((K, D), jnp.float32),
            pltpu.VMEM((ACC_ROWS * D,), jnp.float32),
            pltpu.VMEM((16,), jnp.int32),
            pltpu.SemaphoreType.DMA,
            pltpu.SemaphoreType.DMA,
        ],
    )


def _sc_spmm(*args):
    return _make_sc_spmm()(*args)


def _sc_spmm_body(src_hbm, pmix_hbm, soff_hbm, zeros_hbm, out_hbm,
                  mix0, mix1, rows0, rows1, acc, m16_v, sem0, sem1):
    c = lax.axis_index("c")
    s = lax.axis_index("s")
    w = s * 2 + c

    def load_mix(ch, mix_ref):
        off = pl.multiple_of(2 * K * ch, 2 * K)
        pltpu.sync_copy(pmix_hbm.at[pl.ds(off, 2 * K)], mix_ref)

    def start_gather(mix_ref, rows_ref, sem):
        pltpu.async_copy(src_hbm.at[mix_ref.at[pl.ds(0, K)]], rows_ref, sem)

    def wait_gather(mix_ref, rows_ref, sem):
        pltpu.make_async_copy(src_hbm.at[mix_ref.at[pl.ds(0, K)]],
                              rows_ref, sem).wait()

    def process(mix_ref, rows_ref):
        def grp_body(gg, carry):
            dvec = mix_ref[pl.ds(pl.multiple_of(K + gg * 16, 16), 16)]
            for i16 in range(16):
                base = dvec[i16] * D
                row = gg * 16 + i16
                vals = [rows_ref[row, pl.ds(jj * 16, 16)]
                        for jj in range(D // 16)]
                for jj in range(D // 16):
                    plsc.addupdate(
                        acc.at[pl.ds(pl.multiple_of(base + jj * 16, 16),
                                     16)], vals[jj])
            return carry

        lax.fori_loop(0, K // 16, grp_body, 0)

    for k in range(2):
        j = 2 * w + k
        pltpu.sync_copy(soff_hbm.at[pl.ds(pl.multiple_of(8 * j, 8), 16)],
                        m16_v)
        wv = m16_v[...]
        ch0 = wv[0] // K
        m = (wv[8] - wv[0]) // K
        pltpu.sync_copy(zeros_hbm, acc)

        @pl.when(m > 0)
        def _():
            load_mix(ch0, mix0)
            start_gather(mix0, rows0, sem0)

        def pair_body(ii, carry):
            a = 2 * ii

            @pl.when(a + 1 < m)
            def _():
                load_mix(ch0 + a + 1, mix1)
                start_gather(mix1, rows1, sem1)

            wait_gather(mix0, rows0, sem0)
            process(mix0, rows0)

            @pl.when(a + 1 < m)
            def _():
                @pl.when(a + 2 < m)
                def _():
                    load_mix(ch0 + a + 2, mix0)
                    start_gather(mix0, rows0, sem0)

                wait_gather(mix1, rows1, sem1)
                process(mix1, rows1)

            return carry

        lax.fori_loop(0, (m + 1) // 2, pair_body, 0)
        pltpu.sync_copy(acc.at[pl.ds(0, SEG * D)],
                        out_hbm.at[pl.ds(j * (SEG * D), SEG * D)])


def _build_edge_layout(dst, src):
    order = jnp.argsort(dst)
    dsts = dst[order]
    srcs = src[order]
    seg = dsts // SEG
    cnt = jnp.bincount(seg, length=NSEG)
    goff = jnp.concatenate([jnp.zeros((1,), jnp.int32),
                            jnp.cumsum(cnt).astype(jnp.int32)])
    cnt_p = ((cnt + K - 1) // K) * K
    goff_p = jnp.concatenate([jnp.zeros((1,), jnp.int32),
                              jnp.cumsum(cnt_p).astype(jnp.int32)])
    pos = goff_p[seg] + (jnp.arange(E, dtype=jnp.int32) - goff[seg])
    psrc = jnp.zeros((PAD_E,), jnp.int32).at[pos].set(srcs)
    pdloc = jnp.full((PAD_E,), DUMMY_ROW, jnp.int32).at[pos].set(
        (dsts - seg * SEG).astype(jnp.int32))
    pmix = jnp.concatenate([psrc.reshape(-1, K), pdloc.reshape(-1, K)],
                           axis=1).reshape(-1)
    soff = jnp.zeros((SOFF_LEN,), jnp.int32).at[
        8 * jnp.arange(NSEG + 1)].set(goff_p)
    return pmix, soff


BR = 1000
_NBC = NCL // BR
_NBL = NLITS // BR
_NBV = NV // BR


def _norm_rows(x):
    m = jnp.mean(x, axis=1, keepdims=True)
    xc = x - m
    var = jnp.sum(xc * xc, axis=1, keepdims=True) / (D - 1)
    return xc / (jnp.sqrt(var) + EPS)


def _cu_body(old_ref, msg_ref, w1a_ref, w1b_ref, b1_ref,
             w2_ref, b2_ref, w3_ref, b3_ref, out_ref):
    old = old_ref[...]
    x = (jnp.dot(old, w1a_ref[...], preferred_element_type=jnp.float32)
         + jnp.dot(msg_ref[...], w1b_ref[...], preferred_element_type=jnp.float32)
         + b1_ref[...])
    x = jnp.maximum(x, 0.0)
    x = jnp.maximum(jnp.dot(x, w2_ref[...], preferred_element_type=jnp.float32)
                    + b2_ref[...], 0.0)
    x = jnp.dot(x, w3_ref[...], preferred_element_type=jnp.float32) + b3_ref[...]
    out_ref[...] = _norm_rows(x) + old


def _lu_body(old_ref, msg_ref, flip_ref, w1a_ref, w1b_ref, w1c_ref, b1_ref,
             w2_ref, b2_ref, w3_ref, b3_ref, out_ref):
    old = old_ref[...]
    x = (jnp.dot(old, w1a_ref[...], preferred_element_type=jnp.float32)
         + jnp.dot(msg_ref[...], w1b_ref[...], preferred_element_type=jnp.float32)
         + jnp.dot(flip_ref[...], w1c_ref[...], preferred_element_type=jnp.float32)
         + b1_ref[...])
    x = jnp.maximum(x, 0.0)
    x = jnp.maximum(jnp.dot(x, w2_ref[...], preferred_element_type=jnp.float32)
                    + b2_ref[...], 0.0)
    x = jnp.dot(x, w3_ref[...], preferred_element_type=jnp.float32) + b3_ref[...]
    out_ref[...] = _norm_rows(x) + old


def _vs_body(la_ref, lb_ref, w1a_ref, w1b_ref, b1_ref,
             w2_ref, b2_ref, w3_ref, b3_ref, out_ref):
    x = (jnp.dot(la_ref[...], w1a_ref[...], preferred_element_type=jnp.float32)
         + jnp.dot(lb_ref[...], w1b_ref[...], preferred_element_type=jnp.float32)
         + b1_ref[...])
    x = jnp.maximum(x, 0.0)
    x = jnp.maximum(jnp.dot(x, w2_ref[...], preferred_element_type=jnp.float32)
                    + b2_ref[...], 0.0)
    out_ref[...] = (jnp.dot(x, w3_ref[...], preferred_element_type=jnp.float32)
                    + b3_ref[...])


def _w_spec():
    return pl.BlockSpec((D, D), lambda i: (0, 0))


def _b_spec():
    return pl.BlockSpec((1, D), lambda i: (0, 0))


def _row_spec(idx_fn=None):
    return pl.BlockSpec((BR, D), idx_fn or (lambda i: (i, 0)))


def _call_cu(C, msgs, w1a, w1b, b1, w2, b2, w3, b3):
    return pl.pallas_call(
        _cu_body,
        grid=(_NBC,),
        in_specs=[_row_spec(), _row_spec(), _w_spec(), _w_spec(), _b_spec(),
                  _w_spec(), _b_spec(), _w_spec(), _b_spec()],
        out_specs=_row_spec(),
        out_shape=jax.ShapeDtypeStruct((NCL, D), jnp.float32),
    )(C, msgs, w1a, w1b, b1, w2, b2, w3, b3)


def _call_lu(L, msgs, w1a, w1b, w1c, b1, w2, b2, w3, b3):
    flip_spec = pl.BlockSpec((BR, D), lambda i: ((i + _NBL // 2) % _NBL, 0))
    return pl.pallas_call(
        _lu_body,
        grid=(_NBL,),
        in_specs=[_row_spec(), _row_spec(), flip_spec,
                  _w_spec(), _w_spec(), _w_spec(), _b_spec(),
                  _w_spec(), _b_spec(), _w_spec(), _b_spec()],
        out_specs=_row_spec(),
        out_shape=jax.ShapeDtypeStruct((NLITS, D), jnp.float32),
    )(L, msgs, L, w1a, w1b, w1c, b1, w2, b2, w3, b3)


def _call_vs(L, w1a, w1b, b1, w2, b2, w3p, b3p):
    lb_spec = pl.BlockSpec((BR, D), lambda i: (i + _NBV, 0))
    return pl.pallas_call(
        _vs_body,
        grid=(_NBV,),
        in_specs=[_row_spec(), lb_spec, _w_spec(), _w_spec(), _b_spec(),
                  _w_spec(), _b_spec(),
                  pl.BlockSpec((D, 128), lambda i: (0, 0)),
                  pl.BlockSpec((1, 128), lambda i: (0, 0))],
        out_specs=pl.BlockSpec((BR, 128), lambda i: (i, 0)),
        out_shape=jax.ShapeDtypeStruct((NV, 128), jnp.float32),
    )(L, L, w1a, w1b, b1, w2, b2, w3p, b3p)


def kernel(theta, CL_idxs, n_vars, n_clauses):
    zero = (jnp.asarray(n_vars) - NV
            + jnp.asarray(n_clauses) - NCL).astype(jnp.float32)
    c_idx = CL_idxs[:, 0].astype(jnp.int32)
    l_idx = CL_idxs[:, 1].astype(jnp.int32)

    lc_layout = _build_edge_layout(c_idx, l_idx)
    cl_layout = _build_edge_layout(l_idx, c_idx)
    zeros_acc = jnp.zeros((ACC_ROWS * D,), jnp.float32)

    L = jnp.full((NLITS, D), 1.0, jnp.float32) * theta['L_init'] + zero
    C = jnp.full((NCL, D), 1.0, jnp.float32) * theta['C_init'] + zero

    lc_s = theta['LC']
    cl_s = theta['CL']

    for t in range(8):
        (w1, b1), (w2, b2), (w3, b3) = theta['Cu'][t]
        msgs = _sc_spmm(L, *lc_layout, zeros_acc).reshape(OUT_PAD, D)
        C = _call_cu(C, msgs, w1[:D], w1[D:] * lc_s, b1.reshape(1, D),
                     w2, b2.reshape(1, D), w3, b3.reshape(1, D))
        (w1, b1), (w2, b2), (w3, b3) = theta['Lu'][t]
        msgsl = _sc_spmm(C, *cl_layout, zeros_acc).reshape(OUT_PAD, D)
        L = _call_lu(L, msgsl, w1[:D], w1[D:2 * D] * cl_s, w1[2 * D:],
                     b1.reshape(1, D), w2, b2.reshape(1, D),
                     w3, b3.reshape(1, D))

    (w1, b1), (w2, b2), (w3, b3) = theta['Vs']
    w3p = jnp.zeros((D, 128), jnp.float32).at[:, :1].set(w3)
    b3p = jnp.zeros((1, 128), jnp.float32).at[0, :1].set(b3)
    out = _call_vs(L, w1[:D], w1[D:], b1.reshape(1, D),
                   w2, b2.reshape(1, D), w3p, b3p)
    return out[:, 0]
